# baseline (speedup 1.0000x reference)
import functools

import jax
import jax.numpy as jnp
from jax.experimental import pallas as pl


def _body(x_ref, emb_ref, o_ref, *, n):
    e = emb_ref[...]
    for j in range(n):
        o_ref[:, j, :] = x_ref[:, j, :] + e


def kernel(x, emb):
    L, N, D = x.shape
    BL = 128
    grid = (L // BL,)
    return pl.pallas_call(
        functools.partial(_body, n=N),
        grid=grid,
        in_specs=[
            pl.BlockSpec((BL, N, D), lambda i: (i, 0, 0)),
            pl.BlockSpec((BL, D), lambda i: (i, 0)),
        ],
        out_specs=pl.BlockSpec((BL, N, D), lambda i: (i, 0, 0)),
        out_shape=jax.ShapeDtypeStruct((L, N, D), x.dtype),
    )(x, emb)
